# Initial kernel scaffold; baseline (speedup 1.0000x reference)
import jax
import jax.numpy as jnp
from jax.experimental import pallas as pl


def kernel(p, edge_index, fc_pos_W, fc_pos_b, b0_fc0_W, b0_fc0_b, b0_fc1_W, b0_fc1_b, b0_sc_W, b1_fc0_W, b1_fc0_b, b1_fc1_W, b1_fc1_b, b1_sc_W, b2_fc0_W, b2_fc0_b, b2_fc1_W, b2_fc1_b, b2_sc_W, b3_fc0_W, b3_fc0_b, b3_fc1_W, b3_fc1_b, b3_sc_W, b4_fc0_W, b4_fc0_b, b4_fc1_W, b4_fc1_b, b4_sc_W, fc_c_W, fc_c_b):
    raise NotImplementedError("write your pallas kernel here")


# speedup vs baseline: 1.1391x; 1.1391x over previous
import functools

import jax
import jax.numpy as jnp
from jax import lax
from jax.experimental import pallas as pl
from jax.experimental.pallas import tpu as pltpu
from jax.experimental.pallas import tpu_sc as plsc

F32 = jnp.float32
I32 = jnp.int32

H = 64
FP = 160
TW = 256
NB_ROW = 256
EB = 512
GCH = 80
MCH = 128
NSC = 32
NTPN = 320
NP = NSC * NTPN


def _pad2(a, r, c):
    return jnp.pad(a, ((0, r - a.shape[0]), (0, c - a.shape[1])))


def _row(w):
    return pl.BlockSpec((NB_ROW, w), lambda i: (i, 0))


def _full(shape):
    return pl.BlockSpec(shape, lambda i: (0,) * len(shape))


def _dot(a, b):
    return jax.lax.dot_general(a, b, (((1,), (0,)), ((), ())),
                               precision=jax.lax.Precision.HIGHEST)


def _tables(x, w0t_ref, b0_ref, swb_ref, dw_ref, b1_ref, ta_ref, tb_ref,
            scd_ref):
    xr = jnp.maximum(x, 0.0)
    r0b = _dot(xr, w0t_ref[...]) + b0_ref[...]
    hh = _dot(x, swb_ref[...])
    z = jnp.zeros((x.shape[0], TW - FP - H), x.dtype)
    ta_ref[...] = jnp.concatenate([x, r0b, z], axis=1)
    tb_ref[...] = jnp.concatenate([x, hh, z], axis=1)
    scd_ref[...] = _dot(x, dw_ref[...]) + b1_ref[...]


def _k0_body(p_ref, fcw_ref, fcb_ref, w0t_ref, b0_ref, swb_ref, dw_ref,
             b1_ref, ta_ref, tb_ref, scd_ref):
    x0 = _dot(p_ref[...], fcw_ref[...]) + fcb_ref[...]
    _tables(x0, w0t_ref, b0_ref, swb_ref, dw_ref, b1_ref, ta_ref, tb_ref,
            scd_ref)


def _kmid_body(y_ref, scdp_ref, p_ref, w0t_ref, b0_ref, swb_ref, dw_ref,
               b1_ref, ta_ref, tb_ref, scd_ref):
    yr = y_ref[...]
    y = jnp.where(jnp.isfinite(yr), yr + scdp_ref[...], 0.0)
    pooled = jnp.broadcast_to(jnp.max(y, axis=1, keepdims=True), y.shape)
    xn = jnp.concatenate([y, pooled, p_ref[:, : FP - 2 * H]], axis=1)
    _tables(xn, w0t_ref, b0_ref, swb_ref, dw_ref, b1_ref, ta_ref, tb_ref,
            scd_ref)


def _kfin_body(y_ref, scdp_ref, fcc_ref, fccb_ref, c_ref):
    yr = y_ref[...]
    y = jnp.where(jnp.isfinite(yr), yr + scdp_ref[...], 0.0)
    c_ref[...] = _dot(jnp.maximum(y, 0.0), fcc_ref[...]) + fccb_ref[...]


def _call_k0(p_pad, fcw, fcb, bw):
    outs = (
        [jax.ShapeDtypeStruct((NP, TW), F32)] * 2
        + [jax.ShapeDtypeStruct((NP, H), F32)]
    )
    return pl.pallas_call(
        _k0_body,
        grid=(NP // NB_ROW,),
        in_specs=[
            _row(FP), _full((FP, FP)), _full((1, FP)), _full((FP, H)),
            _full((1, H)), _full((FP, H)), _full((FP, H)), _full((1, H)),
        ],
        out_specs=[_row(TW), _row(TW), _row(H)],
        out_shape=outs,
    )(p_pad, fcw, fcb, bw["w0t"], bw["b0"], bw["swb"], bw["dw"], bw["b1"])


def _call_kmid(y, scd, p_pad, bw):
    outs = (
        [jax.ShapeDtypeStruct((NP, TW), F32)] * 2
        + [jax.ShapeDtypeStruct((NP, H), F32)]
    )
    return pl.pallas_call(
        _kmid_body,
        grid=(NP // NB_ROW,),
        in_specs=[
            _row(H), _row(H), _row(FP), _full((FP, H)), _full((1, H)),
            _full((FP, H)), _full((FP, H)), _full((1, H)),
        ],
        out_specs=[_row(TW), _row(TW), _row(H)],
        out_shape=outs,
    )(y, scd, p_pad, bw["w0t"], bw["b0"], bw["swb"], bw["dw"], bw["b1"])


def _call_kfin(y, scd, fcc, fccb):
    return pl.pallas_call(
        _kfin_body,
        grid=(NP // NB_ROW,),
        in_specs=[_row(H), _row(H), _full((H, H)), _full((1, H))],
        out_specs=_row(H),
        out_shape=jax.ShapeDtypeStruct((NP, H), F32),
    )(y, scd, fcc, fccb)


def _edge_body(pa_ref, pb_ref, w0b_ref, w1_ref, m_ref):
    pa = pa_ref[...]
    pb = pb_ref[...]
    d = pb[:, :FP] - pa[:, :FP]
    t0 = _dot(jnp.maximum(d, 0.0), w0b_ref[...]) + pa[:, FP:FP + H]
    m_ref[...] = _dot(jnp.maximum(t0, 0.0), w1_ref[...]) + pb[:, FP:FP + H]


def _call_edge(pa, pb, w0b, w1, ep):
    return pl.pallas_call(
        _edge_body,
        grid=(ep // EB,),
        in_specs=[
            pl.BlockSpec((EB, TW), lambda i: (i, 0)),
            pl.BlockSpec((EB, TW), lambda i: (i, 0)),
            _full((FP, H)), _full((H, H)),
        ],
        out_specs=pl.BlockSpec((EB, H), lambda i: (i, 0)),
        out_shape=jax.ShapeDtypeStruct((ep, H), F32),
    )(pa, pb, w0b, w1)


def _sc_mesh():
    return plsc.VectorSubcoreMesh(core_axis_name="c", subcore_axis_name="s")


def _sc_gather(src_s, dst_s, ta, tb, e, ep):
    ept = e // NSC
    nch = ept // GCH

    def body(srcs_ref, dsts_ref, ta_ref, tb_ref, pa_ref, pb_ref,
             ia, ib, ba, bb, sa, sb):
        wid = lax.axis_index("s") * 2 + lax.axis_index("c")
        ebase = wid * ept

        def chunk(ci, carry):
            base = ebase + ci * GCH
            pltpu.sync_copy(dsts_ref.at[pl.ds(base, GCH)], ia)
            pltpu.sync_copy(srcs_ref.at[pl.ds(base, GCH)], ib)
            ca = pltpu.async_copy(ta_ref.at[ia], ba, sa)
            cb = pltpu.async_copy(tb_ref.at[ib], bb, sb)
            ca.wait()
            cb.wait()
            pltpu.sync_copy(ba, pa_ref.at[pl.ds(base, GCH)])
            pltpu.sync_copy(bb, pb_ref.at[pl.ds(base, GCH)])
            return carry

        lax.fori_loop(0, nch, chunk, 0)

    f = pl.kernel(
        body,
        out_type=(
            jax.ShapeDtypeStruct((ep, TW), F32),
            jax.ShapeDtypeStruct((ep, TW), F32),
        ),
        mesh=_sc_mesh(),
        scratch_types=[
            pltpu.VMEM((GCH,), I32), pltpu.VMEM((GCH,), I32),
            pltpu.VMEM((GCH, TW), F32), pltpu.VMEM((GCH, TW), F32),
            pltpu.SemaphoreType.DMA, pltpu.SemaphoreType.DMA,
        ],
    )
    return f(src_s, dst_s, ta, tb)


def _sc_max(dst_sp, m, offs):
    def body(dsts_ref, m_ref, offs_ref, y_ref, offs_vm, ids_vm, mb, acc, sm):
        neg_inf = jnp.full((16,), -jnp.inf, F32)
        wid = lax.axis_index("s") * 2 + lax.axis_index("c")
        nbase = wid * NTPN
        pltpu.sync_copy(offs_ref, offs_vm.at[pl.ds(0, 40)])

        def initr(i, c):
            for q in range(4):
                acc[i, pl.ds(q * 16, 16)] = neg_inf
            return c

        lax.fori_loop(0, NTPN, initr, 0)

        ov = offs_vm[pl.ds(wid, 16)]
        e0 = ov[0]
        e1 = ov[1]
        e0a = (e0 // 8) * 8
        nch = (e1 - e0a + MCH - 1) // MCH

        def chunk(k, c):
            base = e0a + k * MCH
            pltpu.sync_copy(dsts_ref.at[pl.ds(base, MCH)],
                            ids_vm.at[pl.ds(0, MCH)])
            pltpu.async_copy(m_ref.at[pl.ds(base, MCH)], mb, sm).wait()

            def grp(g, c2):
                idsv = ids_vm[pl.ds(g * 16, 16)] - nbase
                for l in range(16):
                    e = base + g * 16 + l

                    @pl.when((e >= e0) & (e < e1))
                    def _upd():
                        dl = idsv[l]
                        j = g * 16 + l
                        for q in range(4):
                            s = pl.ds(q * 16, 16)
                            acc[dl, s] = jnp.maximum(acc[dl, s], mb[j, s])

                return c2

            lax.fori_loop(0, MCH // 16, grp, 0)
            return c

        lax.fori_loop(0, nch, chunk, 0)
        pltpu.sync_copy(acc, y_ref.at[pl.ds(nbase, NTPN)])

    f = pl.kernel(
        body,
        out_type=jax.ShapeDtypeStruct((NP, H), F32),
        mesh=_sc_mesh(),
        scratch_types=[
            pltpu.VMEM((48,), I32), pltpu.VMEM((MCH + 16,), I32),
            pltpu.VMEM((MCH, H), F32), pltpu.VMEM((NTPN, H), F32),
            pltpu.SemaphoreType.DMA,
        ],
    )
    return f(dst_sp, m, offs)


def kernel(p, edge_index, fc_pos_W, fc_pos_b,
           b0_fc0_W, b0_fc0_b, b0_fc1_W, b0_fc1_b, b0_sc_W,
           b1_fc0_W, b1_fc0_b, b1_fc1_W, b1_fc1_b, b1_sc_W,
           b2_fc0_W, b2_fc0_b, b2_fc1_W, b2_fc1_b, b2_sc_W,
           b3_fc0_W, b3_fc0_b, b3_fc1_W, b3_fc1_b, b3_sc_W,
           b4_fc0_W, b4_fc0_b, b4_fc1_W, b4_fc1_b, b4_sc_W,
           fc_c_W, fc_c_b):
    n = p.shape[0]
    e = edge_index.shape[1]
    ep = e + EB

    src = edge_index[0].astype(I32)
    dst = edge_index[1].astype(I32)
    perm = jnp.argsort(dst)
    dst_s = dst[perm]
    src_s = src[perm]
    bounds = jnp.arange(NSC + 1, dtype=I32) * NTPN
    offs = jnp.searchsorted(dst_s, bounds).astype(I32)
    offs = jnp.pad(offs, (0, 40 - offs.shape[0]))
    dst_sp = jnp.pad(dst_s, (0, ep - e))

    p_pad = jnp.zeros((NP, FP), F32).at[:n, :3].set(p)
    fcw = _pad2(fc_pos_W, FP, FP)
    fcb = jnp.pad(fc_pos_b, (0, FP - fc_pos_b.shape[0])).reshape(1, FP)

    raw = [
        (b0_fc0_W, b0_fc0_b, b0_fc1_W, b0_fc1_b, b0_sc_W),
        (b1_fc0_W, b1_fc0_b, b1_fc1_W, b1_fc1_b, b1_sc_W),
        (b2_fc0_W, b2_fc0_b, b2_fc1_W, b2_fc1_b, b2_sc_W),
        (b3_fc0_W, b3_fc0_b, b3_fc1_W, b3_fc1_b, b3_sc_W),
        (b4_fc0_W, b4_fc0_b, b4_fc1_W, b4_fc1_b, b4_sc_W),
    ]
    bws = []
    for w0, bb0, w1, bb1, sw in raw:
        f = w0.shape[0] // 2
        bws.append({
            "w0t": _pad2(w0[:f], FP, H),
            "w0b": _pad2(w0[f:], FP, H),
            "b0": bb0.reshape(1, H),
            "w1": w1,
            "b1": bb1.reshape(1, H),
            "swb": _pad2(sw[f:], FP, H),
            "dw": _pad2(sw[:f] - sw[f:], FP, H),
        })

    ta, tb, scd = _call_k0(p_pad, fcw, fcb, bws[0])
    for i in range(5):
        pa, pb = _sc_gather(src_s, dst_s, ta, tb, e, ep)
        m = _call_edge(pa, pb, bws[i]["w0b"], bws[i]["w1"], ep)
        y = _sc_max(dst_sp, m, offs)
        if i < 4:
            ta, tb, scd = _call_kmid(y, scd, p_pad, bws[i + 1])
    c = _call_kfin(y, scd, fc_c_W, fc_c_b.reshape(1, H))
    return c[:n]


# speedup vs baseline: 1.3331x; 1.1703x over previous
import functools

import jax
import jax.numpy as jnp
from jax import lax
from jax.experimental import pallas as pl
from jax.experimental.pallas import tpu as pltpu
from jax.experimental.pallas import tpu_sc as plsc

F32 = jnp.float32
I32 = jnp.int32

H = 64
FP = 160
TW = 256
NB_ROW = 256
EB = 512
GCH = 112
MCH = 128
NSC = 32
NTPN = 320
NP = NSC * NTPN


def _pad2(a, r, c):
    return jnp.pad(a, ((0, r - a.shape[0]), (0, c - a.shape[1])))


def _row(w):
    return pl.BlockSpec((NB_ROW, w), lambda i: (i, 0))


def _full(shape):
    return pl.BlockSpec(shape, lambda i: (0,) * len(shape))


def _dot(a, b):
    return jax.lax.dot_general(a, b, (((1,), (0,)), ((), ())),
                               precision=jax.lax.Precision.HIGHEST)


def _tables(x, w0t_ref, b0_ref, swb_ref, dw_ref, b1_ref, ta_ref, tb_ref,
            scd_ref):
    xr = jnp.maximum(x, 0.0)
    r0b = _dot(xr, w0t_ref[...]) + b0_ref[...]
    hh = _dot(x, swb_ref[...])
    z = jnp.zeros((x.shape[0], TW - FP - H), x.dtype)
    ta_ref[...] = jnp.concatenate([x, r0b, z], axis=1)
    tb_ref[...] = jnp.concatenate([x, hh, z], axis=1)
    scd_ref[...] = _dot(x, dw_ref[...]) + b1_ref[...]


def _k0_body(p_ref, fcw_ref, fcb_ref, w0t_ref, b0_ref, swb_ref, dw_ref,
             b1_ref, ta_ref, tb_ref, scd_ref):
    x0 = _dot(p_ref[...], fcw_ref[...]) + fcb_ref[...]
    _tables(x0, w0t_ref, b0_ref, swb_ref, dw_ref, b1_ref, ta_ref, tb_ref,
            scd_ref)


def _kmid_body(y_ref, scdp_ref, p_ref, w0t_ref, b0_ref, swb_ref, dw_ref,
               b1_ref, ta_ref, tb_ref, scd_ref):
    yr = y_ref[...]
    y = jnp.where(jnp.isfinite(yr), yr + scdp_ref[...], 0.0)
    pooled = jnp.broadcast_to(jnp.max(y, axis=1, keepdims=True), y.shape)
    xn = jnp.concatenate([y, pooled, p_ref[:, : FP - 2 * H]], axis=1)
    _tables(xn, w0t_ref, b0_ref, swb_ref, dw_ref, b1_ref, ta_ref, tb_ref,
            scd_ref)


def _kfin_body(y_ref, scdp_ref, fcc_ref, fccb_ref, c_ref):
    yr = y_ref[...]
    y = jnp.where(jnp.isfinite(yr), yr + scdp_ref[...], 0.0)
    c_ref[...] = _dot(jnp.maximum(y, 0.0), fcc_ref[...]) + fccb_ref[...]


def _call_k0(p_pad, fcw, fcb, bw):
    outs = (
        [jax.ShapeDtypeStruct((NP, TW), F32)] * 2
        + [jax.ShapeDtypeStruct((NP, H), F32)]
    )
    return pl.pallas_call(
        _k0_body,
        grid=(NP // NB_ROW,),
        in_specs=[
            _row(FP), _full((FP, FP)), _full((1, FP)), _full((FP, H)),
            _full((1, H)), _full((FP, H)), _full((FP, H)), _full((1, H)),
        ],
        out_specs=[_row(TW), _row(TW), _row(H)],
        out_shape=outs,
    )(p_pad, fcw, fcb, bw["w0t"], bw["b0"], bw["swb"], bw["dw"], bw["b1"])


def _call_kmid(y, scd, p_pad, bw):
    outs = (
        [jax.ShapeDtypeStruct((NP, TW), F32)] * 2
        + [jax.ShapeDtypeStruct((NP, H), F32)]
    )
    return pl.pallas_call(
        _kmid_body,
        grid=(NP // NB_ROW,),
        in_specs=[
            _row(H), _row(H), _row(FP), _full((FP, H)), _full((1, H)),
            _full((FP, H)), _full((FP, H)), _full((1, H)),
        ],
        out_specs=[_row(TW), _row(TW), _row(H)],
        out_shape=outs,
    )(y, scd, p_pad, bw["w0t"], bw["b0"], bw["swb"], bw["dw"], bw["b1"])


def _call_kfin(y, scd, fcc, fccb):
    return pl.pallas_call(
        _kfin_body,
        grid=(NP // NB_ROW,),
        in_specs=[_row(H), _row(H), _full((H, H)), _full((1, H))],
        out_specs=_row(H),
        out_shape=jax.ShapeDtypeStruct((NP, H), F32),
    )(y, scd, fcc, fccb)


def _edge_body(pa_ref, pb_ref, w0b_ref, w1_ref, m_ref):
    pa = pa_ref[...]
    pb = pb_ref[...]
    d = pb[:, :FP] - pa[:, :FP]
    t0 = _dot(jnp.maximum(d, 0.0), w0b_ref[...]) + pa[:, FP:FP + H]
    m_ref[...] = _dot(jnp.maximum(t0, 0.0), w1_ref[...]) + pb[:, FP:FP + H]


def _call_edge(pa, pb, w0b, w1, ep):
    return pl.pallas_call(
        _edge_body,
        grid=(ep // EB,),
        in_specs=[
            pl.BlockSpec((EB, TW), lambda i: (i, 0)),
            pl.BlockSpec((EB, TW), lambda i: (i, 0)),
            _full((FP, H)), _full((H, H)),
        ],
        out_specs=pl.BlockSpec((EB, H), lambda i: (i, 0)),
        out_shape=jax.ShapeDtypeStruct((ep, H), F32),
    )(pa, pb, w0b, w1)


def _sc_mesh():
    return plsc.VectorSubcoreMesh(core_axis_name="c", subcore_axis_name="s")


def _sc_gather(src_s, dst_s, ta, tb, e, ep):
    ept = e // NSC
    nch = (ept + GCH - 1) // GCH
    lastbase = ept - GCH

    def body(srcs_ref, dsts_ref, ta_ref, tb_ref, pa_ref, pb_ref, *scr):
        ias = scr[0:2]
        ibs = scr[2:4]
        bas = scr[4:6]
        bbs = scr[6:8]
        sias = scr[8:10]
        sibs = scr[10:12]
        sgas = scr[12:14]
        sgbs = scr[14:16]
        swas = scr[16:18]
        swbs = scr[18:20]
        wid = lax.axis_index("s") * 2 + lax.axis_index("c")
        ebase = wid * ept

        def cbase(c):
            return ebase + jnp.minimum(c * GCH, lastbase)

        def wait_idx(r, c):
            pltpu.make_async_copy(
                dsts_ref.at[pl.ds(cbase(c), GCH)], ias[r], sias[r]).wait()
            pltpu.make_async_copy(
                srcs_ref.at[pl.ds(cbase(c), GCH)], ibs[r], sibs[r]).wait()

        def wait_gather(r):
            pltpu.make_async_copy(
                ta_ref.at[pl.ds(0, GCH)], bas[r], sgas[r]).wait()
            pltpu.make_async_copy(
                tb_ref.at[pl.ds(0, GCH)], bbs[r], sgbs[r]).wait()

        def wait_write(r):
            pltpu.make_async_copy(
                ta_ref.at[pl.ds(0, GCH)], bas[r], swas[r]).wait()
            pltpu.make_async_copy(
                tb_ref.at[pl.ds(0, GCH)], bbs[r], swbs[r]).wait()

        def start_idx(r, c):
            pltpu.async_copy(dsts_ref.at[pl.ds(cbase(c), GCH)], ias[r],
                             sias[r])
            pltpu.async_copy(srcs_ref.at[pl.ds(cbase(c), GCH)], ibs[r],
                             sibs[r])

        def start_gather(r):
            pltpu.async_copy(ta_ref.at[ias[r]], bas[r], sgas[r])
            pltpu.async_copy(tb_ref.at[ibs[r]], bbs[r], sgbs[r])

        def start_write(r, c):
            pltpu.async_copy(bas[r], pa_ref.at[pl.ds(cbase(c), GCH)],
                             swas[r])
            pltpu.async_copy(bbs[r], pb_ref.at[pl.ds(cbase(c), GCH)],
                             swbs[r])

        start_idx(0, 0)
        start_idx(1, 1)
        wait_idx(0, 0)
        start_gather(0)

        def pair(k, carry):
            for r in (0, 1):
                c = 2 * k + r
                opp = 1 - r

                @pl.when(c + 1 < nch)
                def _launch():
                    @pl.when(c >= 1)
                    def _wfree():
                        wait_write(opp)

                    wait_idx(opp, c + 1)
                    start_gather(opp)

                wait_gather(r)
                start_write(r, c)

                @pl.when(c + 2 < nch)
                def _nextidx():
                    start_idx(r, c + 2)

            return carry

        lax.fori_loop(0, nch // 2, pair, 0)
        wait_write(0)
        wait_write(1)

    f = pl.kernel(
        body,
        out_type=(
            jax.ShapeDtypeStruct((ep, TW), F32),
            jax.ShapeDtypeStruct((ep, TW), F32),
        ),
        mesh=_sc_mesh(),
        scratch_types=(
            [pltpu.VMEM((GCH,), I32)] * 4
            + [pltpu.VMEM((GCH, TW), F32)] * 4
            + [pltpu.SemaphoreType.DMA] * 12
        ),
    )
    return f(src_s, dst_s, ta, tb)


def _sc_max(dst_sp, m, offs):
    def body(dsts_ref, m_ref, offs_ref, y_ref, offs_vm, ids_vm, mb, acc, sm):
        neg_inf = jnp.full((16,), -jnp.inf, F32)
        wid = lax.axis_index("s") * 2 + lax.axis_index("c")
        nbase = wid * NTPN
        pltpu.sync_copy(offs_ref, offs_vm.at[pl.ds(0, 40)])

        def initr(i, c):
            for q in range(4):
                acc[i, pl.ds(q * 16, 16)] = neg_inf
            return c

        lax.fori_loop(0, NTPN, initr, 0)

        ov = offs_vm[pl.ds(wid, 16)]
        e0 = ov[0]
        e1 = ov[1]
        e0a = (e0 // 8) * 8
        nch = (e1 - e0a + MCH - 1) // MCH

        def chunk(k, c):
            base = e0a + k * MCH
            pltpu.sync_copy(dsts_ref.at[pl.ds(base, MCH)],
                            ids_vm.at[pl.ds(0, MCH)])
            pltpu.async_copy(m_ref.at[pl.ds(base, MCH)], mb, sm).wait()

            def grp(g, c2):
                idsv = ids_vm[pl.ds(g * 16, 16)] - nbase
                for l in range(16):
                    e = base + g * 16 + l

                    @pl.when((e >= e0) & (e < e1))
                    def _upd():
                        dl = idsv[l]
                        j = g * 16 + l
                        for q in range(4):
                            s = pl.ds(q * 16, 16)
                            acc[dl, s] = jnp.maximum(acc[dl, s], mb[j, s])

                return c2

            lax.fori_loop(0, MCH // 16, grp, 0)
            return c

        lax.fori_loop(0, nch, chunk, 0)
        pltpu.sync_copy(acc, y_ref.at[pl.ds(nbase, NTPN)])

    f = pl.kernel(
        body,
        out_type=jax.ShapeDtypeStruct((NP, H), F32),
        mesh=_sc_mesh(),
        scratch_types=[
            pltpu.VMEM((48,), I32), pltpu.VMEM((MCH + 16,), I32),
            pltpu.VMEM((MCH, H), F32), pltpu.VMEM((NTPN, H), F32),
            pltpu.SemaphoreType.DMA,
        ],
    )
    return f(dst_sp, m, offs)


def kernel(p, edge_index, fc_pos_W, fc_pos_b,
           b0_fc0_W, b0_fc0_b, b0_fc1_W, b0_fc1_b, b0_sc_W,
           b1_fc0_W, b1_fc0_b, b1_fc1_W, b1_fc1_b, b1_sc_W,
           b2_fc0_W, b2_fc0_b, b2_fc1_W, b2_fc1_b, b2_sc_W,
           b3_fc0_W, b3_fc0_b, b3_fc1_W, b3_fc1_b, b3_sc_W,
           b4_fc0_W, b4_fc0_b, b4_fc1_W, b4_fc1_b, b4_sc_W,
           fc_c_W, fc_c_b):
    n = p.shape[0]
    e = edge_index.shape[1]
    ep = e + EB

    src = edge_index[0].astype(I32)
    dst = edge_index[1].astype(I32)
    perm = jnp.argsort(dst)
    dst_s = dst[perm]
    src_s = src[perm]
    bounds = jnp.arange(NSC + 1, dtype=I32) * NTPN
    offs = jnp.searchsorted(dst_s, bounds).astype(I32)
    offs = jnp.pad(offs, (0, 40 - offs.shape[0]))
    dst_sp = jnp.pad(dst_s, (0, ep - e))

    p_pad = jnp.zeros((NP, FP), F32).at[:n, :3].set(p)
    fcw = _pad2(fc_pos_W, FP, FP)
    fcb = jnp.pad(fc_pos_b, (0, FP - fc_pos_b.shape[0])).reshape(1, FP)

    raw = [
        (b0_fc0_W, b0_fc0_b, b0_fc1_W, b0_fc1_b, b0_sc_W),
        (b1_fc0_W, b1_fc0_b, b1_fc1_W, b1_fc1_b, b1_sc_W),
        (b2_fc0_W, b2_fc0_b, b2_fc1_W, b2_fc1_b, b2_sc_W),
        (b3_fc0_W, b3_fc0_b, b3_fc1_W, b3_fc1_b, b3_sc_W),
        (b4_fc0_W, b4_fc0_b, b4_fc1_W, b4_fc1_b, b4_sc_W),
    ]
    bws = []
    for w0, bb0, w1, bb1, sw in raw:
        f = w0.shape[0] // 2
        bws.append({
            "w0t": _pad2(w0[:f], FP, H),
            "w0b": _pad2(w0[f:], FP, H),
            "b0": bb0.reshape(1, H),
            "w1": w1,
            "b1": bb1.reshape(1, H),
            "swb": _pad2(sw[f:], FP, H),
            "dw": _pad2(sw[:f] - sw[f:], FP, H),
        })

    ta, tb, scd = _call_k0(p_pad, fcw, fcb, bws[0])
    for i in range(5):
        pa, pb = _sc_gather(src_s, dst_s, ta, tb, e, ep)
        m = _call_edge(pa, pb, bws[i]["w0b"], bws[i]["w1"], ep)
        y = _sc_max(dst_sp, m, offs)
        if i < 4:
            ta, tb, scd = _call_kmid(y, scd, p_pad, bws[i + 1])
    c = _call_kfin(y, scd, fc_c_W, fc_c_b.reshape(1, H))
    return c[:n]


# speedup vs baseline: 1.6415x; 1.2314x over previous
import functools

import jax
import jax.numpy as jnp
from jax import lax
from jax.experimental import pallas as pl
from jax.experimental.pallas import tpu as pltpu
from jax.experimental.pallas import tpu_sc as plsc

F32 = jnp.float32
I32 = jnp.int32

H = 64
FP = 160
TW = 256
NB_ROW = 256
EB = 512
GCH = 64
GR = 3
MCH = 128
NSC = 32
NTPN = 320
NP = NSC * NTPN


def _pad2(a, r, c):
    return jnp.pad(a, ((0, r - a.shape[0]), (0, c - a.shape[1])))


def _row(w):
    return pl.BlockSpec((NB_ROW, w), lambda i: (i, 0))


def _full(shape):
    return pl.BlockSpec(shape, lambda i: (0,) * len(shape))


def _dot(a, b):
    return jax.lax.dot_general(a, b, (((1,), (0,)), ((), ())),
                               precision=jax.lax.Precision.HIGHEST)


def _tables(x, w0t_ref, b0_ref, swb_ref, dw_ref, b1_ref, ta_ref, tb_ref,
            scd_ref):
    xr = jnp.maximum(x, 0.0)
    r0b = _dot(xr, w0t_ref[...]) + b0_ref[...]
    hh = _dot(x, swb_ref[...])
    z = jnp.zeros((x.shape[0], TW - FP - H), x.dtype)
    ta_ref[...] = jnp.concatenate([x, r0b, z], axis=1)
    tb_ref[...] = jnp.concatenate([x, hh, z], axis=1)
    scd_ref[...] = _dot(x, dw_ref[...]) + b1_ref[...]


def _k0_body(p_ref, fcw_ref, fcb_ref, w0t_ref, b0_ref, swb_ref, dw_ref,
             b1_ref, ta_ref, tb_ref, scd_ref):
    x0 = _dot(p_ref[...], fcw_ref[...]) + fcb_ref[...]
    _tables(x0, w0t_ref, b0_ref, swb_ref, dw_ref, b1_ref, ta_ref, tb_ref,
            scd_ref)


def _kmid_body(y_ref, scdp_ref, p_ref, w0t_ref, b0_ref, swb_ref, dw_ref,
               b1_ref, ta_ref, tb_ref, scd_ref):
    yr = y_ref[...]
    y = jnp.where(jnp.isfinite(yr), yr + scdp_ref[...], 0.0)
    pooled = jnp.broadcast_to(jnp.max(y, axis=1, keepdims=True), y.shape)
    xn = jnp.concatenate([y, pooled, p_ref[:, : FP - 2 * H]], axis=1)
    _tables(xn, w0t_ref, b0_ref, swb_ref, dw_ref, b1_ref, ta_ref, tb_ref,
            scd_ref)


def _kfin_body(y_ref, scdp_ref, fcc_ref, fccb_ref, c_ref):
    yr = y_ref[...]
    y = jnp.where(jnp.isfinite(yr), yr + scdp_ref[...], 0.0)
    c_ref[...] = _dot(jnp.maximum(y, 0.0), fcc_ref[...]) + fccb_ref[...]


def _call_k0(p_pad, fcw, fcb, bw):
    outs = (
        [jax.ShapeDtypeStruct((NP, TW), F32)] * 2
        + [jax.ShapeDtypeStruct((NP, H), F32)]
    )
    return pl.pallas_call(
        _k0_body,
        grid=(NP // NB_ROW,),
        in_specs=[
            _row(FP), _full((FP, FP)), _full((1, FP)), _full((FP, H)),
            _full((1, H)), _full((FP, H)), _full((FP, H)), _full((1, H)),
        ],
        out_specs=[_row(TW), _row(TW), _row(H)],
        out_shape=outs,
    )(p_pad, fcw, fcb, bw["w0t"], bw["b0"], bw["swb"], bw["dw"], bw["b1"])


def _call_kmid(y, scd, p_pad, bw):
    outs = (
        [jax.ShapeDtypeStruct((NP, TW), F32)] * 2
        + [jax.ShapeDtypeStruct((NP, H), F32)]
    )
    return pl.pallas_call(
        _kmid_body,
        grid=(NP // NB_ROW,),
        in_specs=[
            _row(H), _row(H), _row(FP), _full((FP, H)), _full((1, H)),
            _full((FP, H)), _full((FP, H)), _full((1, H)),
        ],
        out_specs=[_row(TW), _row(TW), _row(H)],
        out_shape=outs,
    )(y, scd, p_pad, bw["w0t"], bw["b0"], bw["swb"], bw["dw"], bw["b1"])


def _call_kfin(y, scd, fcc, fccb):
    return pl.pallas_call(
        _kfin_body,
        grid=(NP // NB_ROW,),
        in_specs=[_row(H), _row(H), _full((H, H)), _full((1, H))],
        out_specs=_row(H),
        out_shape=jax.ShapeDtypeStruct((NP, H), F32),
    )(y, scd, fcc, fccb)


def _edge_body(pa_ref, pb_ref, w0b_ref, w1_ref, m_ref):
    pa = pa_ref[...]
    pb = pb_ref[...]
    d = pb[:, :FP] - pa[:, :FP]
    t0 = jnp.dot(jnp.maximum(d, 0.0), w0b_ref[...]) + pa[:, FP:FP + H]
    m_ref[...] = jnp.dot(jnp.maximum(t0, 0.0), w1_ref[...]) + pb[:, FP:FP + H]


def _call_edge(pa, pb, w0b, w1, ep):
    return pl.pallas_call(
        _edge_body,
        grid=(ep // EB,),
        in_specs=[
            pl.BlockSpec((EB, TW), lambda i: (i, 0)),
            pl.BlockSpec((EB, TW), lambda i: (i, 0)),
            _full((FP, H)), _full((H, H)),
        ],
        out_specs=pl.BlockSpec((EB, H), lambda i: (i, 0)),
        out_shape=jax.ShapeDtypeStruct((ep, H), F32),
    )(pa, pb, w0b, w1)


def _sc_mesh():
    return plsc.VectorSubcoreMesh(core_axis_name="c", subcore_axis_name="s")


def _sc_gather(src_s, dst_s, ta, tb, e, ep):
    ept = e // NSC
    nch = ((ept + GCH - 1) // GCH + GR - 1) // GR * GR
    lastbase = ept - GCH

    def body(srcs_ref, dsts_ref, ta_ref, tb_ref, pa_ref, pb_ref, *scr):
        ias = scr[0:GR]
        ibs = scr[GR:2 * GR]
        bas = scr[2 * GR:3 * GR]
        bbs = scr[3 * GR:4 * GR]
        sias = scr[4 * GR:5 * GR]
        sibs = scr[5 * GR:6 * GR]
        sgas = scr[6 * GR:7 * GR]
        sgbs = scr[7 * GR:8 * GR]
        swas = scr[8 * GR:9 * GR]
        swbs = scr[9 * GR:10 * GR]
        wid = lax.axis_index("s") * 2 + lax.axis_index("c")
        ebase = wid * ept

        def cbase(c):
            return ebase + jnp.minimum(c * GCH, lastbase)

        def wait_idx(r, c):
            pltpu.make_async_copy(
                dsts_ref.at[pl.ds(cbase(c), GCH)], ias[r], sias[r]).wait()
            pltpu.make_async_copy(
                srcs_ref.at[pl.ds(cbase(c), GCH)], ibs[r], sibs[r]).wait()

        def wait_gather(r):
            pltpu.make_async_copy(
                ta_ref.at[pl.ds(0, GCH)], bas[r], sgas[r]).wait()
            pltpu.make_async_copy(
                tb_ref.at[pl.ds(0, GCH)], bbs[r], sgbs[r]).wait()

        def wait_write(r):
            pltpu.make_async_copy(
                ta_ref.at[pl.ds(0, GCH)], bas[r], swas[r]).wait()
            pltpu.make_async_copy(
                tb_ref.at[pl.ds(0, GCH)], bbs[r], swbs[r]).wait()

        def start_idx(r, c):
            pltpu.async_copy(dsts_ref.at[pl.ds(cbase(c), GCH)], ias[r],
                             sias[r])
            pltpu.async_copy(srcs_ref.at[pl.ds(cbase(c), GCH)], ibs[r],
                             sibs[r])

        def start_gather(r):
            pltpu.async_copy(ta_ref.at[ias[r]], bas[r], sgas[r])
            pltpu.async_copy(tb_ref.at[ibs[r]], bbs[r], sgbs[r])

        def start_write(r, c):
            pltpu.async_copy(bas[r], pa_ref.at[pl.ds(cbase(c), GCH)],
                             swas[r])
            pltpu.async_copy(bbs[r], pb_ref.at[pl.ds(cbase(c), GCH)],
                             swbs[r])

        for r in range(GR):
            start_idx(r, r)
        wait_idx(0, 0)
        start_gather(0)

        def group(k, carry):
            for r in range(GR):
                c = GR * k + r
                nxt = (r + 1) % GR

                @pl.when(c + 1 < nch)
                def _launch():
                    @pl.when(c >= GR - 1)
                    def _wfree():
                        wait_write(nxt)

                    wait_idx(nxt, c + 1)
                    start_gather(nxt)

                wait_gather(r)
                start_write(r, c)

                @pl.when(c + GR < nch)
                def _nextidx():
                    start_idx(r, c + GR)

            return carry

        lax.fori_loop(0, nch // GR, group, 0)
        for r in range(GR):
            wait_write(r)

    f = pl.kernel(
        body,
        out_type=(
            jax.ShapeDtypeStruct((ep, TW), F32),
            jax.ShapeDtypeStruct((ep, TW), F32),
        ),
        mesh=_sc_mesh(),
        scratch_types=(
            [pltpu.VMEM((GCH,), I32)] * (2 * GR)
            + [pltpu.VMEM((GCH, TW), F32)] * (2 * GR)
            + [pltpu.SemaphoreType.DMA] * (6 * GR)
        ),
    )
    return f(src_s, dst_s, ta, tb)


def _sc_max(dst_sp, m, offs):
    def body(dsts_ref, m_ref, offs_ref, y_ref, offs_vm, ids0, ids1, mb0,
             mb1, acc, si0, si1, sm0, sm1):
        neg_inf = jnp.full((16,), -jnp.inf, F32)
        wid = lax.axis_index("s") * 2 + lax.axis_index("c")
        nbase = wid * NTPN
        pltpu.sync_copy(offs_ref, offs_vm.at[pl.ds(0, 40)])

        ids_s = (ids0, ids1)
        mb_s = (mb0, mb1)
        si_s = (si0, si1)
        sm_s = (sm0, sm1)

        ov = offs_vm[pl.ds(wid, 16)]
        e0 = ov[0]
        e1 = ov[1]
        e0a = (e0 // 8) * 8
        nch = (e1 - e0a + MCH - 1) // MCH

        def start(c, r):
            base = e0a + c * MCH
            pltpu.async_copy(dsts_ref.at[pl.ds(base, MCH)],
                             ids_s[r].at[pl.ds(0, MCH)], si_s[r])
            pltpu.async_copy(m_ref.at[pl.ds(base, MCH)], mb_s[r], sm_s[r])

        def wait(c, r):
            base = e0a + c * MCH
            pltpu.make_async_copy(
                dsts_ref.at[pl.ds(base, MCH)],
                ids_s[r].at[pl.ds(0, MCH)], si_s[r]).wait()
            pltpu.make_async_copy(
                m_ref.at[pl.ds(base, MCH)], mb_s[r], sm_s[r]).wait()

        def compute(c, r):
            base = e0a + c * MCH
            ids_vm = ids_s[r]
            mb = mb_s[r]

            def grp(g, c2):
                idsv = ids_vm[pl.ds(g * 16, 16)] - nbase
                for l in range(16):
                    e = base + g * 16 + l

                    @pl.when((e >= e0) & (e < e1))
                    def _upd():
                        dl = idsv[l]
                        j = g * 16 + l
                        for q in range(4):
                            s = pl.ds(q * 16, 16)
                            acc[dl, s] = jnp.maximum(acc[dl, s], mb[j, s])

                return c2

            lax.fori_loop(0, MCH // 16, grp, 0)

        @pl.when(nch > 0)
        def _pro():
            start(0, 0)

        def initr(i, c):
            for q in range(4):
                acc[i, pl.ds(q * 16, 16)] = neg_inf
            return c

        lax.fori_loop(0, NTPN, initr, 0)

        def pair(k, c):
            c0 = 2 * k

            @pl.when(c0 + 1 < nch)
            def _s1():
                start(c0 + 1, 1)

            wait(c0, 0)
            compute(c0, 0)

            @pl.when(c0 + 2 < nch)
            def _s0():
                start(c0 + 2, 0)

            @pl.when(c0 + 1 < nch)
            def _c1():
                wait(c0 + 1, 1)
                compute(c0 + 1, 1)

            return c

        lax.fori_loop(0, (nch + 1) // 2, pair, 0)
        pltpu.sync_copy(acc, y_ref.at[pl.ds(nbase, NTPN)])

    f = pl.kernel(
        body,
        out_type=jax.ShapeDtypeStruct((NP, H), F32),
        mesh=_sc_mesh(),
        scratch_types=[
            pltpu.VMEM((48,), I32),
            pltpu.VMEM((MCH + 16,), I32), pltpu.VMEM((MCH + 16,), I32),
            pltpu.VMEM((MCH, H), F32), pltpu.VMEM((MCH, H), F32),
            pltpu.VMEM((NTPN, H), F32),
            pltpu.SemaphoreType.DMA, pltpu.SemaphoreType.DMA,
            pltpu.SemaphoreType.DMA, pltpu.SemaphoreType.DMA,
        ],
    )
    return f(dst_sp, m, offs)


def kernel(p, edge_index, fc_pos_W, fc_pos_b,
           b0_fc0_W, b0_fc0_b, b0_fc1_W, b0_fc1_b, b0_sc_W,
           b1_fc0_W, b1_fc0_b, b1_fc1_W, b1_fc1_b, b1_sc_W,
           b2_fc0_W, b2_fc0_b, b2_fc1_W, b2_fc1_b, b2_sc_W,
           b3_fc0_W, b3_fc0_b, b3_fc1_W, b3_fc1_b, b3_sc_W,
           b4_fc0_W, b4_fc0_b, b4_fc1_W, b4_fc1_b, b4_sc_W,
           fc_c_W, fc_c_b):
    n = p.shape[0]
    e = edge_index.shape[1]
    ep = e + EB

    src = edge_index[0].astype(I32)
    dst = edge_index[1].astype(I32)
    perm = jnp.argsort(dst)
    dst_s = dst[perm]
    src_s = src[perm]
    bounds = jnp.arange(NSC + 1, dtype=I32) * NTPN
    offs = jnp.searchsorted(dst_s, bounds).astype(I32)
    offs = jnp.pad(offs, (0, 40 - offs.shape[0]))
    dst_sp = jnp.pad(dst_s, (0, ep - e))

    p_pad = jnp.zeros((NP, FP), F32).at[:n, :3].set(p)
    fcw = _pad2(fc_pos_W, FP, FP)
    fcb = jnp.pad(fc_pos_b, (0, FP - fc_pos_b.shape[0])).reshape(1, FP)

    raw = [
        (b0_fc0_W, b0_fc0_b, b0_fc1_W, b0_fc1_b, b0_sc_W),
        (b1_fc0_W, b1_fc0_b, b1_fc1_W, b1_fc1_b, b1_sc_W),
        (b2_fc0_W, b2_fc0_b, b2_fc1_W, b2_fc1_b, b2_sc_W),
        (b3_fc0_W, b3_fc0_b, b3_fc1_W, b3_fc1_b, b3_sc_W),
        (b4_fc0_W, b4_fc0_b, b4_fc1_W, b4_fc1_b, b4_sc_W),
    ]
    bws = []
    for w0, bb0, w1, bb1, sw in raw:
        f = w0.shape[0] // 2
        bws.append({
            "w0t": _pad2(w0[:f], FP, H),
            "w0b": _pad2(w0[f:], FP, H),
            "b0": bb0.reshape(1, H),
            "w1": w1,
            "b1": bb1.reshape(1, H),
            "swb": _pad2(sw[f:], FP, H),
            "dw": _pad2(sw[:f] - sw[f:], FP, H),
        })

    ta, tb, scd = _call_k0(p_pad, fcw, fcb, bws[0])
    for i in range(5):
        pa, pb = _sc_gather(src_s, dst_s, ta, tb, e, ep)
        m = _call_edge(pa, pb, bws[i]["w0b"], bws[i]["w1"], ep)
        y = _sc_max(dst_sp, m, offs)
        if i < 4:
            ta, tb, scd = _call_kmid(y, scd, p_pad, bws[i + 1])
    c = _call_kfin(y, scd, fc_c_W, fc_c_b.reshape(1, H))
    return c[:n]
